# baseline (speedup 1.0000x reference)
import functools

import jax
import jax.numpy as jnp
from jax import lax
from jax.experimental import pallas as pl
from jax.experimental.pallas import tpu as pltpu
from jax.experimental.pallas import tpu_sc as plsc


def _make_sc_gather(BSZ, H, V, D):
    info = plsc.get_sparse_core_info()
    nc, ns = info.num_cores, info.num_subcores
    nw = nc * ns
    assert BSZ % nw == 0
    rows_per_w = BSZ // nw
    mesh = plsc.VectorSubcoreMesh(core_axis_name="c", subcore_axis_name="s")

    @functools.partial(
        pl.kernel,
        mesh=mesh,
        compiler_params=pltpu.CompilerParams(use_tc_tiling_on_sc=False),
        out_type=jax.ShapeDtypeStruct((BSZ, H, D), jnp.float32),
        scratch_types=[
            pltpu.VMEM_SHARED((V, D), jnp.float32),
            pltpu.VMEM((H,), jnp.int32),
            pltpu.VMEM((H,), jnp.int32),
            pltpu.VMEM((H, D), jnp.float32),
            pltpu.VMEM((H, D), jnp.float32),
            pltpu.SemaphoreType.DMA,
            pltpu.SemaphoreType.DMA,
            pltpu.SemaphoreType.DMA,
            pltpu.SemaphoreType.DMA,
        ],
    )
    def gather_kernel(idx_hbm, table_hbm, out_hbm,
                      table_sh, idx0, idx1, rows0, rows1,
                      sg0, sg1, so0, so1):
        sid = lax.axis_index("s")
        wid = sid * nc + lax.axis_index("c")
        bb = wid * rows_per_w

        @pl.when(sid == 0)
        def _():
            pltpu.sync_copy(table_hbm, table_sh)

        plsc.subcore_barrier()

        idx_v = (idx0, idx1)
        rows_v = (rows0, rows1)
        sg = (sg0, sg1)
        so = (so0, so1)

        def load_and_gather(i, b):
            pltpu.sync_copy(idx_hbm.at[pl.ds((bb + i) * H, H)], idx_v[b])
            pltpu.make_async_copy(
                table_sh.at[idx_v[b]], rows_v[b], sg[b]).start()

        def wait_gather_start_store(i, b):
            pltpu.make_async_copy(
                table_sh.at[idx_v[b]], rows_v[b], sg[b]).wait()
            pltpu.make_async_copy(
                rows_v[b], out_hbm.at[bb + i], so[b]).start()

        def wait_store(i, b):
            pltpu.make_async_copy(
                rows_v[b], out_hbm.at[bb + i], so[b]).wait()

        load_and_gather(0, 0)
        load_and_gather(1, 1)
        wait_gather_start_store(0, 0)

        def body(j, carry):
            i0 = 2 * j + 2
            wait_store(i0 - 2, 0)
            load_and_gather(i0, 0)
            wait_gather_start_store(i0 - 1, 1)
            wait_store(i0 - 1, 1)
            load_and_gather(i0 + 1, 1)
            wait_gather_start_store(i0, 0)
            return carry

        lax.fori_loop(0, rows_per_w // 2 - 1, body, 0)

        last = rows_per_w - 1
        wait_gather_start_store(last, 1)
        wait_store(last - 1, 0)
        wait_store(last, 1)

    return gather_kernel


def kernel(indices, position_emb):
    bsz, hist = indices.shape
    v, d = position_emb.shape
    flat_idx = indices.reshape(bsz * hist)
    fn = _make_sc_gather(bsz, hist, v, d)
    return fn(flat_idx, position_emb)


# speedup vs baseline: 1.5905x; 1.5905x over previous
import functools

import jax
import jax.numpy as jnp
from jax import lax
from jax.experimental import pallas as pl
from jax.experimental.pallas import tpu as pltpu
from jax.experimental.pallas import tpu_sc as plsc
from jax.experimental.layout import Format, Layout, with_layout_constraint


def _make_sc_gather(B, V, D, chunk):
    info = plsc.get_sparse_core_info()
    nc, ns = info.num_cores, info.num_subcores
    nw = nc * ns
    assert B % nw == 0
    b_per_w = B // nw
    assert b_per_w % chunk == 0
    n_chunks = b_per_w // chunk
    mesh = plsc.VectorSubcoreMesh(core_axis_name="c", subcore_axis_name="s")

    @functools.partial(
        pl.kernel,
        mesh=mesh,
        compiler_params=pltpu.CompilerParams(use_tc_tiling_on_sc=False),
        out_type=jax.ShapeDtypeStruct((B, D), jnp.float32),
        scratch_types=[
            pltpu.VMEM_SHARED((V, D), jnp.float32),
            pltpu.VMEM((chunk,), jnp.int32),
            pltpu.VMEM((chunk,), jnp.int32),
            pltpu.VMEM((chunk, D), jnp.float32),
            pltpu.VMEM((chunk, D), jnp.float32),
            pltpu.SemaphoreType.DMA,
            pltpu.SemaphoreType.DMA,
            pltpu.SemaphoreType.DMA,
            pltpu.SemaphoreType.DMA,
        ],
    )
    def gather_kernel(idx_hbm, table_hbm, out_hbm,
                      table_sh, idx0, idx1, rows0, rows1, sg0, sg1, so0, so1):
        sid = lax.axis_index("s")
        wid = sid * nc + lax.axis_index("c")
        base = wid * b_per_w

        @pl.when(sid == 0)
        def _():
            pltpu.sync_copy(table_hbm, table_sh)

        plsc.subcore_barrier()

        idx_v = (idx0, idx1)
        rows_v = (rows0, rows1)
        sg = (sg0, sg1)
        so = (so0, so1)
        gathers = [None, None]
        stores = [None, None]
        for i in range(n_chunks):
            b = i & 1
            if stores[b] is not None:
                stores[b].wait()
            off = base + i * chunk
            pltpu.sync_copy(idx_hbm.at[pl.ds(off, chunk)], idx_v[b])
            gathers[b] = pltpu.async_copy(
                table_sh.at[idx_v[b]], rows_v[b], sg[b])
            if i >= 1:
                pb = 1 - b
                gathers[pb].wait()
                stores[pb] = pltpu.async_copy(
                    rows_v[pb],
                    out_hbm.at[pl.ds(base + (i - 1) * chunk, chunk)],
                    so[pb])
        lb = (n_chunks - 1) & 1
        gathers[lb].wait()
        stores[lb] = pltpu.async_copy(
            rows_v[lb],
            out_hbm.at[pl.ds(base + (n_chunks - 1) * chunk, chunk)],
            so[lb])
        if stores[1 - lb] is not None:
            stores[1 - lb].wait()
        stores[lb].wait()

    return gather_kernel


def kernel(indices, position_emb):
    bsz, hist = indices.shape
    v, d = position_emb.shape
    flat_idx = indices.reshape(bsz * hist)
    fn = _make_sc_gather(bsz * hist, v, d, 1600)
    out = fn(flat_idx, position_emb)
    return with_layout_constraint(
        out.reshape(bsz, hist, d),
        Layout(major_to_minor=(0, 1, 2), tiling=((8,),)))


# speedup vs baseline: 1.5934x; 1.0018x over previous
import functools

import jax
import jax.numpy as jnp
from jax import lax
from jax.experimental import pallas as pl
from jax.experimental.pallas import tpu as pltpu
from jax.experimental.pallas import tpu_sc as plsc
from jax.experimental.layout import Format, Layout, with_layout_constraint


def _make_sc_gather(B, V, D, chunk):
    info = plsc.get_sparse_core_info()
    nc, ns = info.num_cores, info.num_subcores
    nw = nc * ns
    assert B % nw == 0
    b_per_w = B // nw
    assert b_per_w % chunk == 0
    n_chunks = b_per_w // chunk
    mesh = plsc.VectorSubcoreMesh(core_axis_name="c", subcore_axis_name="s")

    @functools.partial(
        pl.kernel,
        mesh=mesh,
        compiler_params=pltpu.CompilerParams(use_tc_tiling_on_sc=False),
        out_type=jax.ShapeDtypeStruct((B, D), jnp.float32),
        scratch_types=[
            pltpu.VMEM_SHARED((V, D), jnp.float32),
            pltpu.VMEM((chunk,), jnp.int32),
            pltpu.VMEM((chunk,), jnp.int32),
            pltpu.VMEM((chunk, D), jnp.float32),
            pltpu.VMEM((chunk, D), jnp.float32),
            pltpu.SemaphoreType.DMA,
            pltpu.SemaphoreType.DMA,
            pltpu.SemaphoreType.DMA,
            pltpu.SemaphoreType.DMA,
        ],
    )
    def gather_kernel(idx_hbm, table_hbm, out_hbm,
                      table_sh, idx0, idx1, rows0, rows1, sg0, sg1, so0, so1):
        sid = lax.axis_index("s")
        wid = sid * nc + lax.axis_index("c")
        base = wid * b_per_w

        @pl.when(sid == 0)
        def _():
            pltpu.sync_copy(table_hbm, table_sh)

        plsc.subcore_barrier()

        idx_v = (idx0, idx1)
        rows_v = (rows0, rows1)
        sg = (sg0, sg1)
        so = (so0, so1)
        gathers = [None, None]
        stores = [None, None]
        for i in range(n_chunks):
            b = i & 1
            if stores[b] is not None:
                stores[b].wait()
            off = base + i * chunk
            pltpu.sync_copy(idx_hbm.at[pl.ds(off, chunk)], idx_v[b])
            gathers[b] = pltpu.async_copy(
                table_sh.at[idx_v[b]], rows_v[b], sg[b])
            if i >= 1:
                pb = 1 - b
                gathers[pb].wait()
                stores[pb] = pltpu.async_copy(
                    rows_v[pb],
                    out_hbm.at[pl.ds(base + (i - 1) * chunk, chunk)],
                    so[pb])
        lb = (n_chunks - 1) & 1
        gathers[lb].wait()
        stores[lb] = pltpu.async_copy(
            rows_v[lb],
            out_hbm.at[pl.ds(base + (n_chunks - 1) * chunk, chunk)],
            so[lb])
        if stores[1 - lb] is not None:
            stores[1 - lb].wait()
        stores[lb].wait()

    return gather_kernel


def kernel(indices, position_emb):
    bsz, hist = indices.shape
    v, d = position_emb.shape
    flat_idx = indices.reshape(bsz * hist)
    fn = _make_sc_gather(bsz * hist, v, d, 1600)
    out = fn(flat_idx, position_emb)
    return with_layout_constraint(
        out.reshape(bsz, hist, d),
        Layout(major_to_minor=(0, 1, 2), tiling=((8, 128),)))
